# baseline (speedup 1.0000x reference)
import functools

import jax
import jax.numpy as jnp
from jax import lax
from jax.experimental import pallas as pl
from jax.experimental.pallas import tpu as pltpu
from jax.experimental.pallas import tpu_sc as plsc

_MARGIN = 0.5
_NON_MATCH_LOSS_WEIGHT = 1.0
_L = 16


def _sc_geometry():
    try:
        info = plsc.get_sparse_core_info()
        return info.num_cores, info.num_subcores
    except Exception:
        return 2, 16


@functools.partial(jax.jit, static_argnums=(2, 3, 4))
def _repack(ta, tb, B, N, D):
    NC, NS = _sc_geometry()
    NW = NC * NS
    P2 = B * N // 2
    PPW = P2 // NW
    CH1 = 112
    NCH1 = PPW // CH1
    assert CH1 * NCH1 == PPW
    WPB = NW // B
    SRW = 2 * PPW
    mesh = plsc.VectorSubcoreMesh(core_axis_name="c", subcore_axis_name="s",
                                  num_cores=NC, num_subcores=NS)

    def body(ta_hbm, tb_hbm, pta_hbm, ptb_hbm, rbuf, wbuf, semr, semw):
        wid = lax.axis_index("s") * NC + lax.axis_index("c")
        b_t = wid // WPB
        src0 = (wid % WPB) * SRW
        dst0 = wid * PPW

        def fire_read(t, ch, p):
            src = ta_hbm if t == 0 else tb_hbm
            r0 = src0 + ch * 2 * CH1
            return pltpu.async_copy(src.at[b_t, pl.ds(r0, 2 * CH1), :],
                                    rbuf[p], semr[p])

        def fire_write(t, ch, p):
            dst = pta_hbm if t == 0 else ptb_hbm
            d0 = dst0 + ch * CH1
            return pltpu.async_copy(wbuf[p], dst.at[pl.ds(d0, CH1), :],
                                    semw[p])

        steps = [(t, ch) for t in range(2) for ch in range(NCH1)]
        n = len(steps)
        rd = [None, None]
        wr = [None, None]
        rd[0] = fire_read(*steps[0], 0)
        rd[1] = fire_read(*steps[1], 1)
        for k in range(n):
            p = k & 1
            rd[p].wait()
            if wr[p] is not None:
                wr[p].wait()
            def pk(i, carry):
                for cc in range(D // _L):
                    sl = pl.ds(cc * _L, _L)
                    wbuf[p][i, pl.ds(cc * _L, _L)] = rbuf[p][2 * i, sl]
                    wbuf[p][i, pl.ds(D + cc * _L, _L)] = rbuf[p][2 * i + 1, sl]
                return carry
            lax.fori_loop(0, CH1, pk, jnp.int32(0))
            wr[p] = fire_write(*steps[k], p)
            if k + 2 < n:
                rd[p] = fire_read(*steps[k + 2], p)
        wr[0].wait()
        wr[1].wait()

    call = pl.kernel(
        body,
        out_type=(jax.ShapeDtypeStruct((P2, 2 * D), jnp.float32),
                  jax.ShapeDtypeStruct((P2, 2 * D), jnp.float32)),
        mesh=mesh,
        scratch_types=[
            [pltpu.VMEM((2 * CH1, D), jnp.float32) for _ in range(2)],
            [pltpu.VMEM((CH1, 2 * D), jnp.float32) for _ in range(2)],
            [pltpu.SemaphoreType.DMA for _ in range(2)],
            [pltpu.SemaphoreType.DMA for _ in range(2)],
        ],
    )
    return call(ta, tb)


@functools.partial(jax.jit, static_argnums=(6, 7, 8, 9))
def _partials(pta, ptb, mA, mB, nA, nB, B, N, D, M):
    NC, NS = _sc_geometry()
    NW = NC * NS
    PW = M // NW
    CH = min(PW, 128)
    NCH = PW // CH
    CC = D // _L
    G = CH // _L
    N2 = N // 2
    mesh = plsc.VectorSubcoreMesh(core_axis_name="c", subcore_axis_name="s",
                                  num_cores=NC, num_subcores=NS)
    n_chunks = B * 2 * NCH

    def body(ta_hbm, tb_hbm, mA_hbm, mB_hbm, nA_hbm, nB_hbm, out_hbm,
             idxa, idxb, rowa, rowb, rowsa, rowsb, res_v, semi, semr):
        wid = lax.axis_index("s") * NC + lax.axis_index("c")
        base = wid * PW

        def chunk_desc(k):
            b, r = divmod(k, 2 * NCH)
            si, ch = divmod(r, NCH)
            return b, si, ch

        def fire_idx(k, p):
            b, si, ch = chunk_desc(k)
            iA = mA_hbm if si == 0 else nA_hbm
            iB = mB_hbm if si == 0 else nB_hbm
            start = b * M + base + ch * CH
            ca = pltpu.async_copy(iA.at[pl.ds(start, CH)],
                                  idxa[p].at[pl.ds(0, CH)], semi[p])
            cb = pltpu.async_copy(iB.at[pl.ds(start, CH)],
                                  idxb[p].at[pl.ds(0, CH)], semi[p])
            return ca, cb

        def prep(k, p):
            b, _, _ = chunk_desc(k)
            roff = jnp.int32(b * N2)
            for i in range(G):
                sl = pl.ds(i * _L, _L)
                rowa[p][sl] = (idxa[p][sl] >> 1) + roff
                rowb[p][sl] = (idxb[p][sl] >> 1) + roff

        def fire_rows(p):
            ca = pltpu.async_copy(ta_hbm.at[rowa[p]], rowsa[p], semr[p])
            cb = pltpu.async_copy(tb_hbm.at[rowb[p]], rowsb[p], semr[p])
            return ca, cb

        idx_cps = [None, None]
        row_cps = [None, None]
        idx_cps[0] = fire_idx(0, 0)
        idx_cps[1] = fire_idx(1, 1)
        idx_cps[0][0].wait()
        idx_cps[0][1].wait()
        prep(0, 0)
        row_cps[0] = fire_rows(0)

        acc_m = acc_p = acc_c = None

        for k in range(n_chunks):
            p = k & 1
            q = (k + 1) & 1
            b, si, ch = chunk_desc(k)
            row_cps[p][0].wait()
            row_cps[p][1].wait()
            if k + 1 < n_chunks:
                idx_cps[q][0].wait()
                idx_cps[q][1].wait()
                prep(k + 1, q)
                row_cps[q] = fire_rows(q)
            if ch == 0:
                if si == 0:
                    acc_m = [jnp.zeros((_L,), jnp.float32) for _ in range(CC)]
                else:
                    acc_p = [jnp.zeros((_L,), jnp.float32) for _ in range(CC)]
                    acc_c = [jnp.zeros((_L,), jnp.float32) for _ in range(CC)]
            if si == 0:
                def rbody_m(r, accs):
                    accs = list(accs)
                    ha = (idxa[p][pl.ds(r, _L)][0] & 1) * D
                    hb = (idxb[p][pl.ds(r, _L)][0] & 1) * D
                    for cc in range(CC):
                        av = rowsa[p][r, pl.ds(ha + cc * _L, _L)]
                        bv = rowsb[p][r, pl.ds(hb + cc * _L, _L)]
                        d = av - bv
                        accs[cc] = accs[cc] + d * d
                    return tuple(accs)
                acc_m = list(lax.fori_loop(0, CH, rbody_m, tuple(acc_m)))
            else:
                def rbody_n(r, accs):
                    a0 = list(accs[0])
                    a1 = list(accs[1])
                    ha = (idxa[p][pl.ds(r, _L)][0] & 1) * D
                    hb = (idxb[p][pl.ds(r, _L)][0] & 1) * D
                    for cc in range(CC):
                        av = rowsa[p][r, pl.ds(ha + cc * _L, _L)]
                        bv = rowsb[p][r, pl.ds(hb + cc * _L, _L)]
                        d = av - bv
                        t = _MARGIN - d * d
                        pos = t > 0.0
                        a0[cc] = a0[cc] + jnp.where(pos, t, 0.0)
                        a1[cc] = a1[cc] + jnp.where(pos, 1.0, 0.0)
                    return tuple(a0), tuple(a1)
                acc_p, acc_c = lax.fori_loop(0, CH, rbody_n,
                                             (tuple(acc_p), tuple(acc_c)))
                acc_p = list(acc_p)
                acc_c = list(acc_c)
            if ch == NCH - 1:
                if si == 0:
                    res_v[pl.ds(b * 64, _L)] = (acc_m[0] + acc_m[1]) + (acc_m[2] + acc_m[3])
                else:
                    res_v[pl.ds(b * 64 + _L, _L)] = (acc_p[0] + acc_p[1]) + (acc_p[2] + acc_p[3])
                    res_v[pl.ds(b * 64 + 2 * _L, _L)] = (acc_c[0] + acc_c[1]) + (acc_c[2] + acc_c[3])
            if k + 2 < n_chunks:
                idx_cps[p] = fire_idx(k + 2, p)

        pltpu.sync_copy(res_v, out_hbm.at[pl.ds(wid * 4 * B * _L, 4 * B * _L)])

    call = pl.kernel(
        body,
        out_type=jax.ShapeDtypeStruct((NW * B * 4 * _L,), jnp.float32),
        mesh=mesh,
        scratch_types=[
            [pltpu.VMEM((CH + _L,), jnp.int32) for _ in range(2)],
            [pltpu.VMEM((CH + _L,), jnp.int32) for _ in range(2)],
            [pltpu.VMEM((CH,), jnp.int32) for _ in range(2)],
            [pltpu.VMEM((CH,), jnp.int32) for _ in range(2)],
            [pltpu.VMEM((CH, 2 * D), jnp.float32) for _ in range(2)],
            [pltpu.VMEM((CH, 2 * D), jnp.float32) for _ in range(2)],
            pltpu.VMEM((B * 4 * _L,), jnp.float32),
            [pltpu.SemaphoreType.DMA for _ in range(2)],
            [pltpu.SemaphoreType.DMA for _ in range(2)],
        ],
    )
    return call(pta, ptb, mA, mB, nA, nB)


def kernel(outA, outB, matchA, matchB, nonMatchA, nonMatchB):
    B, N, D = outA.shape
    M = matchA.shape[1]
    mA = matchA.astype(jnp.int32).reshape(-1)
    mB = matchB.astype(jnp.int32).reshape(-1)
    nA = nonMatchA.astype(jnp.int32).reshape(-1)
    nB = nonMatchB.astype(jnp.int32).reshape(-1)
    pta, ptb = _repack(outA, outB, B, N, D)
    parts = _partials(pta, ptb, mA, mB, nA, nB, B, N, D, M)
    NC, NS = _sc_geometry()
    sums = jnp.sum(parts.reshape(NC * NS, B, 4, _L), axis=(0, 3))
    match_loss = jnp.sum(sums[:, 0]) / M
    non_match_loss = _NON_MATCH_LOSS_WEIGHT * jnp.sum(sums[:, 1] / sums[:, 2])
    return (match_loss + non_match_loss, match_loss, non_match_loss)
